# baseline (speedup 1.0000x reference)
import functools

import jax
import jax.numpy as jnp
from jax import lax
from jax.experimental import pallas as pl
from jax.experimental.pallas import tpu as pltpu
from jax.experimental.pallas import tpu_sc as plsc

B, C, H, W = 16, 3, 224, 224
PATCH = 16
D = 768
NQ = 100
NC = 4
E = 3
P = (H // PATCH) * (W // PATCH)
PD = C * PATCH * PATCH


def _patchify(x, p=PATCH):
    b, c, h, w = x.shape
    x = x.reshape(b, c, h // p, p, w // p, p)
    x = x.transpose(0, 2, 4, 1, 3, 5)
    return x.reshape(b, (h // p) * (w // p), c * p * p)


def _route_logits_body(x_ref, wc_ref, bc_ref, out_ref):
    pooled = jnp.sum(x_ref[...], axis=(2, 3)) * (1.0 / (H * W))
    logits = jnp.dot(pooled, wc_ref[...],
                     preferred_element_type=jnp.float32) + bc_ref[...]
    out_ref[...] = logits.T


def _route_logits(pixel_values, Wc, bc_row):
    return pl.pallas_call(
        _route_logits_body,
        out_shape=jax.ShapeDtypeStruct((E, B), jnp.float32),
    )(pixel_values, Wc, bc_row)


def _route_sc_body(logits_hbm, perm_hbm, ech_hbm, lv, pv, ev, kv):
    cid = lax.axis_index("c")
    sid = lax.axis_index("s")

    @pl.when(jnp.logical_and(cid == 0, sid == 0))
    def _():
        pltpu.sync_copy(logits_hbm, lv)
        l0 = lv[0, :]
        l1 = lv[1, :]
        l2 = lv[2, :]
        one = jnp.full((B,), 1, jnp.int32)
        two = jnp.full((B,), 2, jnp.int32)
        zero = jnp.full((B,), 0, jnp.int32)
        c = jnp.where(l1 > l0, one, zero)
        c = jnp.where(l2 > jnp.maximum(l0, l1), two, c)
        idx = lax.iota(jnp.int32, B)
        key = c * B + idx
        kv[...] = key
        rank = zero
        for s in range(1, B):
            rot = lax.bitwise_and(idx + s, B - 1)
            ks = plsc.load_gather(kv, [rot])
            rank = rank + jnp.where(ks < key, one, zero)
        plsc.store_scatter(pv, [rank], idx)
        plsc.store_scatter(ev, [rank], c)
        pltpu.sync_copy(pv, perm_hbm)
        pltpu.sync_copy(ev, ech_hbm)


def _route_sc(logitsT):
    mesh = plsc.VectorSubcoreMesh(core_axis_name="c", subcore_axis_name="s")
    fn = functools.partial(
        pl.kernel,
        out_type=(jax.ShapeDtypeStruct((B,), jnp.int32),
                  jax.ShapeDtypeStruct((B,), jnp.int32)),
        mesh=mesh,
        scratch_types=[pltpu.VMEM((E, B), jnp.float32),
                       pltpu.VMEM((B,), jnp.int32),
                       pltpu.VMEM((B,), jnp.int32),
                       pltpu.VMEM((B,), jnp.int32)],
        compiler_params=pltpu.CompilerParams(needs_layout_passes=False),
    )(_route_sc_body)
    return fn(logitsT)


def _moe_body(pm, ec, px_ref, wp_ref, q_ref, wk_ref, wv_ref,
              wcls_ref, bcls_ref, wbox_ref, bbox_ref,
              logits_ref, boxes_ref, hidden_ref, buf_ref):
    del pm
    i = pl.program_id(0)
    e = ec[jnp.maximum(i - 1, 0)]
    x = buf_ref[...].astype(jnp.float32)
    tokens = jnp.dot(x, wp_ref[e], preferred_element_type=jnp.float32)
    k = jnp.dot(tokens, wk_ref[e], preferred_element_type=jnp.float32)
    v = jnp.dot(tokens, wv_ref[e], preferred_element_type=jnp.float32)
    scores = lax.dot_general(q_ref[e], k, (((1,), (1,)), ((), ())),
                             preferred_element_type=jnp.float32)
    scores = scores * (1.0 / jnp.sqrt(jnp.float32(D)))
    attn = jax.nn.softmax(scores, axis=-1)
    hidden = jnp.dot(attn, v, preferred_element_type=jnp.float32)
    hidden_ref[0] = hidden
    logits_ref[0] = jnp.dot(hidden, wcls_ref[e],
                            preferred_element_type=jnp.float32) + bcls_ref[e]
    boxes_ref[0] = jax.nn.sigmoid(
        jnp.dot(hidden, wbox_ref[e],
                preferred_element_type=jnp.float32) + bbox_ref[e])
    NP = H // PATCH
    xb = px_ref[0].astype(jnp.bfloat16)
    chans = [
        xb[c].reshape(NP, PATCH, NP, PATCH)
        .transpose(0, 2, 1, 3).reshape(P, PATCH * PATCH)
        for c in range(C)
    ]
    buf_ref[...] = jnp.concatenate(chans, axis=1)


def _moe_grid_spec():
    img = lambda i, pm, ec: (pm[jnp.maximum(i - 1, 0)], 0, 0)
    full = lambda i, pm, ec: (0, 0, 0)
    return pltpu.PrefetchScalarGridSpec(
        num_scalar_prefetch=2,
        grid=(B + 1,),
        in_specs=[
            pl.BlockSpec((1, C, H, W),
                         lambda i, pm, ec: (pm[jnp.minimum(i, B - 1)], 0, 0, 0)),
            pl.BlockSpec((E, PD, D), full),
            pl.BlockSpec((E, NQ, D), full),
            pl.BlockSpec((E, D, D), full),
            pl.BlockSpec((E, D, D), full),
            pl.BlockSpec((E, D, NC), full),
            pl.BlockSpec((E, 1, NC), full),
            pl.BlockSpec((E, D, 4), full),
            pl.BlockSpec((E, 1, 4), full),
        ],
        out_specs=[
            pl.BlockSpec((1, NQ, NC), img),
            pl.BlockSpec((1, NQ, 4), img),
            pl.BlockSpec((1, NQ, D), img),
        ],
        scratch_shapes=[pltpu.VMEM((P, PD), jnp.bfloat16)],
    )


def _moe_call(perm, ech, pixel_values, Wp, Q, Wk, Wv, Wcls, bcls3, Wbox, bbox3):
    return pl.pallas_call(
        _moe_body,
        grid_spec=_moe_grid_spec(),
        out_shape=[
            jax.ShapeDtypeStruct((B, NQ, NC), jnp.float32),
            jax.ShapeDtypeStruct((B, NQ, 4), jnp.float32),
            jax.ShapeDtypeStruct((B, NQ, D), jnp.float32),
        ],
        compiler_params=pltpu.CompilerParams(
            dimension_semantics=("arbitrary",)),
    )(perm, ech, pixel_values, Wp, Q, Wk, Wv, Wcls, bcls3, Wbox, bbox3)


def kernel(pixel_values, Wc, bc, Wp, Q, Wk, Wv, Wcls, bcls, Wbox, bbox):
    logitsT = _route_logits(pixel_values, Wc, bc.reshape(1, E))
    perm, ech = _route_sc(logitsT)
    logits, boxes, hidden = _moe_call(
        perm, ech, pixel_values, Wp, Q, Wk, Wv, Wcls,
        bcls.reshape(E, 1, NC), Wbox, bbox.reshape(E, 1, 4))
    return logits, boxes, hidden


# speedup vs baseline: 1.0541x; 1.0541x over previous
import functools

import jax
import jax.numpy as jnp
from jax import lax
from jax.experimental import pallas as pl
from jax.experimental.pallas import tpu as pltpu
from jax.experimental.pallas import tpu_sc as plsc

B, C, H, W = 16, 3, 224, 224
PATCH = 16
D = 768
NQ = 100
NC = 4
E = 3
P = (H // PATCH) * (W // PATCH)
PD = C * PATCH * PATCH


def _patchify(x, p=PATCH):
    b, c, h, w = x.shape
    x = x.reshape(b, c, h // p, p, w // p, p)
    x = x.transpose(0, 2, 4, 1, 3, 5)
    return x.reshape(b, (h // p) * (w // p), c * p * p)


def _route_logits_body(x_ref, wc_ref, bc_ref, out_ref):
    pooled = jnp.sum(x_ref[...], axis=(2, 3)) * (1.0 / (H * W))
    logits = jnp.dot(pooled, wc_ref[...],
                     preferred_element_type=jnp.float32) + bc_ref[...]
    out_ref[...] = logits.T


def _route_logits(pixel_values, Wc, bc_row):
    return pl.pallas_call(
        _route_logits_body,
        out_shape=jax.ShapeDtypeStruct((E, B), jnp.float32),
    )(pixel_values, Wc, bc_row)


def _route_sc_body(logits_hbm, perm_hbm, ech_hbm, lv, pv, ev, kv):
    cid = lax.axis_index("c")
    sid = lax.axis_index("s")

    @pl.when(jnp.logical_and(cid == 0, sid == 0))
    def _():
        pltpu.sync_copy(logits_hbm, lv)
        l0 = lv[0, :]
        l1 = lv[1, :]
        l2 = lv[2, :]
        one = jnp.full((B,), 1, jnp.int32)
        two = jnp.full((B,), 2, jnp.int32)
        zero = jnp.full((B,), 0, jnp.int32)
        c = jnp.where(l1 > l0, one, zero)
        c = jnp.where(l2 > jnp.maximum(l0, l1), two, c)
        idx = lax.iota(jnp.int32, B)
        key = c * B + idx
        kv[...] = key
        rank = zero
        for s in range(1, B):
            rot = lax.bitwise_and(idx + s, B - 1)
            ks = plsc.load_gather(kv, [rot])
            rank = rank + jnp.where(ks < key, one, zero)
        plsc.store_scatter(pv, [rank], idx)
        plsc.store_scatter(ev, [rank], c)
        pltpu.sync_copy(pv, perm_hbm)
        pltpu.sync_copy(ev, ech_hbm)


def _route_sc(logitsT):
    mesh = plsc.VectorSubcoreMesh(core_axis_name="c", subcore_axis_name="s")
    fn = functools.partial(
        pl.kernel,
        out_type=(jax.ShapeDtypeStruct((B,), jnp.int32),
                  jax.ShapeDtypeStruct((B,), jnp.int32)),
        mesh=mesh,
        scratch_types=[pltpu.VMEM((E, B), jnp.float32),
                       pltpu.VMEM((B,), jnp.int32),
                       pltpu.VMEM((B,), jnp.int32),
                       pltpu.VMEM((B,), jnp.int32)],
        compiler_params=pltpu.CompilerParams(needs_layout_passes=False),
    )(_route_sc_body)
    return fn(logitsT)


def _moe_body(pm, ec, px_ref, wp_ref, q_ref, wk_ref, wv_ref,
              wcls_ref, bcls_ref, wbox_ref, bbox_ref,
              logits_ref, boxes_ref, hidden_ref, buf_ref,
              wpb_ref, wkb_ref, wvb_ref, qb_ref):
    del pm
    i = pl.program_id(0)
    e_now = ec[jnp.maximum(i - 1, 0)]
    e_prev = ec[jnp.maximum(i - 2, 0)]

    @pl.when(jnp.logical_or(i == 0, e_now != e_prev))
    def _cast_weights():
        wpb_ref[...] = wp_ref[0].astype(jnp.bfloat16)
        wkb_ref[...] = wk_ref[0].astype(jnp.bfloat16)
        wvb_ref[...] = wv_ref[0].astype(jnp.bfloat16)
        qb_ref[...] = (q_ref[0] * (1.0 / jnp.sqrt(jnp.float32(D)))
                       ).astype(jnp.bfloat16)

    x = buf_ref[...]
    tokens = jnp.dot(x, wpb_ref[...], preferred_element_type=jnp.float32)
    tokens_b = tokens.astype(jnp.bfloat16)
    k = jnp.dot(tokens_b, wkb_ref[...], preferred_element_type=jnp.float32)
    v = jnp.dot(tokens_b, wvb_ref[...], preferred_element_type=jnp.float32)
    scores = lax.dot_general(qb_ref[...], k.astype(jnp.bfloat16),
                             (((1,), (1,)), ((), ())),
                             preferred_element_type=jnp.float32)
    attn = jax.nn.softmax(scores, axis=-1)
    hidden = jnp.dot(attn.astype(jnp.bfloat16), v.astype(jnp.bfloat16),
                     preferred_element_type=jnp.float32)
    hidden_ref[0] = hidden
    logits_ref[0] = jnp.dot(hidden, wcls_ref[0],
                            preferred_element_type=jnp.float32) + bcls_ref[0]
    boxes_ref[0] = jax.nn.sigmoid(
        jnp.dot(hidden, wbox_ref[0],
                preferred_element_type=jnp.float32) + bbox_ref[0])
    NP = H // PATCH
    xb = px_ref[0].astype(jnp.bfloat16)
    chans = [
        xb[c].reshape(NP, PATCH, NP, PATCH)
        .transpose(0, 2, 1, 3).reshape(P, PATCH * PATCH)
        for c in range(C)
    ]
    buf_ref[...] = jnp.concatenate(chans, axis=1)


def _moe_grid_spec():
    img = lambda i, pm, ec: (pm[jnp.maximum(i - 1, 0)], 0, 0)
    exp = lambda i, pm, ec: (ec[jnp.maximum(i - 1, 0)], 0, 0)
    return pltpu.PrefetchScalarGridSpec(
        num_scalar_prefetch=2,
        grid=(B + 1,),
        in_specs=[
            pl.BlockSpec((1, C, H, W),
                         lambda i, pm, ec: (pm[jnp.minimum(i, B - 1)], 0, 0, 0)),
            pl.BlockSpec((1, PD, D), exp),
            pl.BlockSpec((1, NQ, D), exp),
            pl.BlockSpec((1, D, D), exp),
            pl.BlockSpec((1, D, D), exp),
            pl.BlockSpec((1, D, NC), exp),
            pl.BlockSpec((1, 1, NC), exp),
            pl.BlockSpec((1, D, 4), exp),
            pl.BlockSpec((1, 1, 4), exp),
        ],
        out_specs=[
            pl.BlockSpec((1, NQ, NC), img),
            pl.BlockSpec((1, NQ, 4), img),
            pl.BlockSpec((1, NQ, D), img),
        ],
        scratch_shapes=[pltpu.VMEM((P, PD), jnp.bfloat16),
                        pltpu.VMEM((PD, D), jnp.bfloat16),
                        pltpu.VMEM((D, D), jnp.bfloat16),
                        pltpu.VMEM((D, D), jnp.bfloat16),
                        pltpu.VMEM((NQ, D), jnp.bfloat16)],
    )


def _moe_call(perm, ech, pixel_values, Wp, Q, Wk, Wv, Wcls, bcls3, Wbox, bbox3):
    return pl.pallas_call(
        _moe_body,
        grid_spec=_moe_grid_spec(),
        out_shape=[
            jax.ShapeDtypeStruct((B, NQ, NC), jnp.float32),
            jax.ShapeDtypeStruct((B, NQ, 4), jnp.float32),
            jax.ShapeDtypeStruct((B, NQ, D), jnp.float32),
        ],
        compiler_params=pltpu.CompilerParams(
            dimension_semantics=("arbitrary",)),
    )(perm, ech, pixel_values, Wp, Q, Wk, Wv, Wcls, bcls3, Wbox, bbox3)


def kernel(pixel_values, Wc, bc, Wp, Q, Wk, Wv, Wcls, bcls, Wbox, bbox):
    logitsT = _route_logits(pixel_values, Wc, bc.reshape(1, E))
    perm, ech = _route_sc(logitsT)
    logits, boxes, hidden = _moe_call(
        perm, ech, pixel_values, Wp, Q, Wk, Wv, Wcls,
        bcls.reshape(E, 1, NC), Wbox, bbox.reshape(E, 1, 4))
    return logits, boxes, hidden


# speedup vs baseline: 1.1303x; 1.0723x over previous
import functools

import jax
import jax.numpy as jnp
from jax import lax
from jax.experimental import pallas as pl
from jax.experimental.pallas import tpu as pltpu
from jax.experimental.pallas import tpu_sc as plsc

B, C, H, W = 16, 3, 224, 224
PATCH = 16
D = 768
NQ = 100
NC = 4
E = 3
P = (H // PATCH) * (W // PATCH)
PD = C * PATCH * PATCH


def _patchify(x, p=PATCH):
    b, c, h, w = x.shape
    x = x.reshape(b, c, h // p, p, w // p, p)
    x = x.transpose(0, 2, 4, 1, 3, 5)
    return x.reshape(b, (h // p) * (w // p), c * p * p)


def _route_logits_body(x_ref, wc_ref, bc_ref, out_ref):
    pooled = jnp.sum(x_ref[...], axis=(2, 3)) * (1.0 / (H * W))
    logits = jnp.dot(pooled, wc_ref[...],
                     preferred_element_type=jnp.float32) + bc_ref[...]
    out_ref[...] = logits.T


def _route_logits(pixel_values, Wc, bc_row):
    return pl.pallas_call(
        _route_logits_body,
        out_shape=jax.ShapeDtypeStruct((E, B), jnp.float32),
    )(pixel_values, Wc, bc_row)


def _route_sc_body(logits_hbm, pmec_hbm, lv, pv, ev, kv):
    cid = lax.axis_index("c")
    sid = lax.axis_index("s")

    @pl.when(jnp.logical_and(cid == 0, sid == 0))
    def _():
        pltpu.sync_copy(logits_hbm, lv)
        l0 = lv[0, :]
        l1 = lv[1, :]
        l2 = lv[2, :]
        one = jnp.full((B,), 1, jnp.int32)
        two = jnp.full((B,), 2, jnp.int32)
        zero = jnp.full((B,), 0, jnp.int32)
        c = jnp.where(l1 > l0, one, zero)
        c = jnp.where(l2 > jnp.maximum(l0, l1), two, c)
        idx = lax.iota(jnp.int32, B)
        key = c * B + idx
        kv[...] = key
        rank = zero
        for s in range(1, B):
            rot = lax.bitwise_and(idx + s, B - 1)
            ks = plsc.load_gather(kv, [rot])
            rank = rank + jnp.where(ks < key, one, zero)
        plsc.store_scatter(pv, [rank], idx)
        plsc.store_scatter(ev, [rank], c)
        pltpu.sync_copy(pv, pmec_hbm.at[0])
        pltpu.sync_copy(ev, pmec_hbm.at[1])


def _route_sc(logitsT):
    mesh = plsc.VectorSubcoreMesh(core_axis_name="c", subcore_axis_name="s")
    fn = functools.partial(
        pl.kernel,
        out_type=jax.ShapeDtypeStruct((2, B), jnp.int32),
        mesh=mesh,
        scratch_types=[pltpu.VMEM((E, B), jnp.float32),
                       pltpu.VMEM((B,), jnp.int32),
                       pltpu.VMEM((B,), jnp.int32),
                       pltpu.VMEM((B,), jnp.int32)],
        compiler_params=pltpu.CompilerParams(needs_layout_passes=False),
    )(_route_sc_body)
    return fn(logitsT)


def _moe_body(pmec, px_ref, wp_ref, q_ref, wk_ref, wv_ref,
              wcls_ref, bcls_ref, wbox_ref, bbox_ref,
              logits_ref, boxes_ref, hidden_ref, buf_ref,
              wpb_ref, wkb_ref, wvb_ref, qb_ref):
    i = pl.program_id(0)
    e_cur = pmec[1, jnp.maximum(i - 1, 0)]
    e_next = pmec[1, jnp.minimum(i, B - 1)]

    x = buf_ref[...]
    tokens = jnp.dot(x, wpb_ref[...], preferred_element_type=jnp.float32)
    tokens_b = tokens.astype(jnp.bfloat16)
    k = jnp.dot(tokens_b, wkb_ref[...], preferred_element_type=jnp.float32)
    v = jnp.dot(tokens_b, wvb_ref[...], preferred_element_type=jnp.float32)
    scores = lax.dot_general(qb_ref[...], k.astype(jnp.bfloat16),
                             (((1,), (1,)), ((), ())),
                             preferred_element_type=jnp.float32)
    attn = jax.nn.softmax(scores, axis=-1)
    hidden = jnp.dot(attn.astype(jnp.bfloat16), v.astype(jnp.bfloat16),
                     preferred_element_type=jnp.float32)
    hidden_ref[0] = hidden
    logits_ref[0] = (jnp.dot(hidden, wcls_ref[0],
                             preferred_element_type=jnp.float32)
                     + bcls_ref[0]).T
    boxes_ref[0] = jax.nn.sigmoid(
        jnp.dot(hidden, wbox_ref[0],
                preferred_element_type=jnp.float32) + bbox_ref[0]).T
    NP = H // PATCH
    xb = px_ref[0].astype(jnp.bfloat16)
    chans = [
        xb[c].reshape(NP, PATCH, NP, PATCH)
        .transpose(0, 2, 1, 3).reshape(P, PATCH * PATCH)
        for c in range(C)
    ]
    buf_ref[...] = jnp.concatenate(chans, axis=1)

    @pl.when(jnp.logical_or(i == 0, e_next != e_cur))
    def _cast_weights():
        wpb_ref[...] = wp_ref[0].astype(jnp.bfloat16)
        wkb_ref[...] = wk_ref[0].astype(jnp.bfloat16)
        wvb_ref[...] = wv_ref[0].astype(jnp.bfloat16)
        qb_ref[...] = (q_ref[0] * (1.0 / jnp.sqrt(jnp.float32(D)))
                       ).astype(jnp.bfloat16)


def _moe_grid_spec():
    img = lambda i, pmec: (pmec[0, jnp.maximum(i - 1, 0)], 0, 0)
    exp = lambda i, pmec: (pmec[1, jnp.maximum(i - 1, 0)], 0, 0)
    expa = lambda i, pmec: (pmec[1, jnp.minimum(i, B - 1)], 0, 0)
    return pltpu.PrefetchScalarGridSpec(
        num_scalar_prefetch=1,
        grid=(B + 1,),
        in_specs=[
            pl.BlockSpec((1, C, H, W),
                         lambda i, pmec: (pmec[0, jnp.minimum(i, B - 1)], 0, 0, 0)),
            pl.BlockSpec((1, PD, D), expa),
            pl.BlockSpec((1, NQ, D), expa),
            pl.BlockSpec((1, D, D), expa),
            pl.BlockSpec((1, D, D), expa),
            pl.BlockSpec((1, D, NC), exp),
            pl.BlockSpec((1, 1, NC), exp),
            pl.BlockSpec((1, D, 4), exp),
            pl.BlockSpec((1, 1, 4), exp),
        ],
        out_specs=[
            pl.BlockSpec((1, NC, NQ), img),
            pl.BlockSpec((1, 4, NQ), img),
            pl.BlockSpec((1, NQ, D), img),
        ],
        scratch_shapes=[pltpu.VMEM((P, PD), jnp.bfloat16),
                        pltpu.VMEM((PD, D), jnp.bfloat16),
                        pltpu.VMEM((D, D), jnp.bfloat16),
                        pltpu.VMEM((D, D), jnp.bfloat16),
                        pltpu.VMEM((NQ, D), jnp.bfloat16)],
    )


def _moe_call(pmec, pixel_values, Wp, Q, Wk, Wv, Wcls, bcls, Wbox, bbox):
    return pl.pallas_call(
        _moe_body,
        grid_spec=_moe_grid_spec(),
        out_shape=[
            jax.ShapeDtypeStruct((B, NC, NQ), jnp.float32),
            jax.ShapeDtypeStruct((B, 4, NQ), jnp.float32),
            jax.ShapeDtypeStruct((B, NQ, D), jnp.float32),
        ],
        compiler_params=pltpu.CompilerParams(
            dimension_semantics=("arbitrary",)),
    )(pmec, pixel_values, Wp, Q, Wk, Wv, Wcls, bcls, Wbox, bbox)


def kernel(pixel_values, Wc, bc, Wp, Q, Wk, Wv, Wcls, bcls, Wbox, bbox):
    logitsT = _route_logits(pixel_values, Wc, bc.reshape(1, E))
    pmec = _route_sc(logitsT)
    logits_t, boxes_t, hidden = _moe_call(
        pmec, pixel_values, Wp, Q, Wk, Wv, Wcls,
        bcls.reshape(E, 1, NC), Wbox, bbox.reshape(E, 1, 4))
    return (jnp.transpose(logits_t, (0, 2, 1)),
            jnp.transpose(boxes_t, (0, 2, 1)), hidden)


# speedup vs baseline: 1.1311x; 1.0007x over previous
import functools

import jax
import jax.numpy as jnp
from jax import lax
from jax.experimental import pallas as pl
from jax.experimental.pallas import tpu as pltpu
from jax.experimental.pallas import tpu_sc as plsc

B, C, H, W = 16, 3, 224, 224
PATCH = 16
D = 768
NQ = 100
NC = 4
E = 3
P = (H // PATCH) * (W // PATCH)
PD = C * PATCH * PATCH


def _patchify(x, p=PATCH):
    b, c, h, w = x.shape
    x = x.reshape(b, c, h // p, p, w // p, p)
    x = x.transpose(0, 2, 4, 1, 3, 5)
    return x.reshape(b, (h // p) * (w // p), c * p * p)


def _route_logits_body(x_ref, wc_ref, bc_ref, out_ref):
    pooled = jnp.sum(x_ref[...], axis=(2, 3)) * (1.0 / (H * W))
    logits = jnp.dot(pooled, wc_ref[...],
                     preferred_element_type=jnp.float32) + bc_ref[...]
    out_ref[...] = logits.T


def _route_logits(pixel_values, Wc, bc_row):
    return pl.pallas_call(
        _route_logits_body,
        out_shape=jax.ShapeDtypeStruct((E, B), jnp.float32),
    )(pixel_values, Wc, bc_row)


def _route_sc_body(logits_hbm, pmec_hbm, lv, pv, ev, kv):
    cid = lax.axis_index("c")
    sid = lax.axis_index("s")

    @pl.when(jnp.logical_and(cid == 0, sid == 0))
    def _():
        pltpu.sync_copy(logits_hbm, lv)
        l0 = lv[0, :]
        l1 = lv[1, :]
        l2 = lv[2, :]
        one = jnp.full((B,), 1, jnp.int32)
        two = jnp.full((B,), 2, jnp.int32)
        zero = jnp.full((B,), 0, jnp.int32)
        c = jnp.where(l1 > l0, one, zero)
        c = jnp.where(l2 > jnp.maximum(l0, l1), two, c)
        idx = lax.iota(jnp.int32, B)
        key = c * B + idx
        kv[...] = key
        rank = zero
        for s in range(1, B):
            rot = lax.bitwise_and(idx + s, B - 1)
            ks = plsc.load_gather(kv, [rot])
            rank = rank + jnp.where(ks < key, one, zero)
        plsc.store_scatter(pv, [rank], idx)
        plsc.store_scatter(ev, [rank], c)
        pltpu.sync_copy(pv, pmec_hbm.at[0])
        pltpu.sync_copy(ev, pmec_hbm.at[1])


def _route_sc(logitsT):
    mesh = plsc.VectorSubcoreMesh(core_axis_name="c", subcore_axis_name="s")
    fn = functools.partial(
        pl.kernel,
        out_type=jax.ShapeDtypeStruct((2, B), jnp.int32),
        mesh=mesh,
        scratch_types=[pltpu.VMEM((E, B), jnp.float32),
                       pltpu.VMEM((B,), jnp.int32),
                       pltpu.VMEM((B,), jnp.int32),
                       pltpu.VMEM((B,), jnp.int32)],
        compiler_params=pltpu.CompilerParams(needs_layout_passes=False),
    )(_route_sc_body)
    return fn(logitsT)


def _moe_body(pmec, px_ref, wp_ref, q_ref, wk_ref, wv_ref,
              wcls_ref, bcls_ref, wbox_ref, bbox_ref,
              logits_ref, boxes_ref, hidden_ref, buf_ref,
              wpb_ref, wkb_ref, wvb_ref, qb_ref):
    i = pl.program_id(0)
    e_now = pmec[1, jnp.maximum(i - 1, 0)]
    e_prev = pmec[1, jnp.maximum(i - 2, 0)]

    @pl.when(jnp.logical_or(i == 0, e_now != e_prev))
    def _cast_weights():
        wpb_ref[...] = wp_ref[0].astype(jnp.bfloat16)
        wkb_ref[...] = wk_ref[0].astype(jnp.bfloat16)
        wvb_ref[...] = wv_ref[0].astype(jnp.bfloat16)
        qb_ref[...] = (q_ref[0] * (1.0 / jnp.sqrt(jnp.float32(D)))
                       ).astype(jnp.bfloat16)

    x = buf_ref[...]
    tokens = jnp.dot(x, wpb_ref[...], preferred_element_type=jnp.float32)
    tokens_b = tokens.astype(jnp.bfloat16)
    k = jnp.dot(tokens_b, wkb_ref[...], preferred_element_type=jnp.float32)
    v = jnp.dot(tokens_b, wvb_ref[...], preferred_element_type=jnp.float32)
    scores = lax.dot_general(qb_ref[...], k.astype(jnp.bfloat16),
                             (((1,), (1,)), ((), ())),
                             preferred_element_type=jnp.float32)
    attn = jax.nn.softmax(scores, axis=-1)
    hidden = jnp.dot(attn.astype(jnp.bfloat16), v.astype(jnp.bfloat16),
                     preferred_element_type=jnp.float32)
    hidden_ref[0] = hidden
    logits_ref[0] = (jnp.dot(hidden, wcls_ref[0],
                             preferred_element_type=jnp.float32)
                     + bcls_ref[0]).T
    boxes_ref[0] = jax.nn.sigmoid(
        jnp.dot(hidden, wbox_ref[0],
                preferred_element_type=jnp.float32) + bbox_ref[0]).T
    NP = H // PATCH
    xb = px_ref[0].astype(jnp.bfloat16)
    chans = [
        xb[c].reshape(NP, PATCH, NP, PATCH)
        .transpose(0, 2, 1, 3).reshape(P, PATCH * PATCH)
        for c in range(C)
    ]
    buf_ref[...] = jnp.concatenate(chans, axis=1)


def _moe_grid_spec():
    img = lambda i, pmec: (pmec[0, jnp.maximum(i - 1, 0)], 0, 0)
    exp = lambda i, pmec: (pmec[1, jnp.maximum(i - 1, 0)], 0, 0)
    exp2 = lambda i, pmec: (pmec[1, jnp.maximum(i - 1, 0)], 0)
    return pltpu.PrefetchScalarGridSpec(
        num_scalar_prefetch=1,
        grid=(B + 1,),
        in_specs=[
            pl.BlockSpec((1, C, H, W),
                         lambda i, pmec: (pmec[0, jnp.minimum(i, B - 1)], 0, 0, 0)),
            pl.BlockSpec((1, PD, D), exp),
            pl.BlockSpec((1, NQ, D), exp),
            pl.BlockSpec((1, D, D), exp),
            pl.BlockSpec((1, D, D), exp),
            pl.BlockSpec((1, D, NC), exp),
            pl.BlockSpec((1, 1, NC), exp),
            pl.BlockSpec((1, D, 4), exp),
            pl.BlockSpec((1, 1, 4), exp),
        ],
        out_specs=[
            pl.BlockSpec((1, NC, NQ), img),
            pl.BlockSpec((1, 4, NQ), img),
            pl.BlockSpec((1, NQ, D), img),
        ],
        scratch_shapes=[pltpu.VMEM((P, PD), jnp.bfloat16),
                        pltpu.VMEM((PD, D), jnp.bfloat16),
                        pltpu.VMEM((D, D), jnp.bfloat16),
                        pltpu.VMEM((D, D), jnp.bfloat16),
                        pltpu.VMEM((NQ, D), jnp.bfloat16)],
    )


def _moe_call(pmec, pixel_values, Wp, Q, Wk, Wv, Wcls, bcls, Wbox, bbox):
    return pl.pallas_call(
        _moe_body,
        grid_spec=_moe_grid_spec(),
        out_shape=[
            jax.ShapeDtypeStruct((B, NC, NQ), jnp.float32),
            jax.ShapeDtypeStruct((B, 4, NQ), jnp.float32),
            jax.ShapeDtypeStruct((B, NQ, D), jnp.float32),
        ],
        compiler_params=pltpu.CompilerParams(
            dimension_semantics=("arbitrary",)),
    )(pmec, pixel_values, Wp, Q, Wk, Wv, Wcls, bcls, Wbox, bbox)


def kernel(pixel_values, Wc, bc, Wp, Q, Wk, Wv, Wcls, bcls, Wbox, bbox):
    logitsT = _route_logits(pixel_values, Wc, bc.reshape(1, E))
    pmec = _route_sc(logitsT)
    logits_t, boxes_t, hidden = _moe_call(
        pmec, pixel_values, Wp, Q, Wk, Wv, Wcls,
        bcls.reshape(E, 1, NC), Wbox, bbox.reshape(E, 1, 4))
    return (jnp.transpose(logits_t, (0, 2, 1)),
            jnp.transpose(boxes_t, (0, 2, 1)), hidden)
